# baseline (device time: 90515 ns/iter reference)
import jax
import jax.numpy as jnp
from jax import lax
from jax.experimental import pallas as pl
from jax.experimental.pallas import tpu as pltpu

N_DEV = 16
_GELU_C = 0.7978845608028654

_SHIFT_ORDER = [8, 7, 9, 6, 10, 5, 11, 4, 12, 3, 13, 2, 14, 1, 15, 0]

_N_WSLOT = 3
_N_SSLOT = 3


def _gelu(y):
    return 0.5 * y * (1.0 + jnp.tanh(_GELU_C * (y + 0.044715 * y * y * y)))


def kernel(x, w_mat):
    m_per, k_dim = x.shape
    _, n = w_mat.shape
    n_per = n // N_DEV

    def body(x_ref, w_hbm, out_ref, x_bf, w_buf, send_buf, recv_buf,
             copy_sems, send_sems, recv_sems):
        me = lax.axis_index("i")

        bar = pltpu.get_barrier_semaphore()
        for kk in range(1, N_DEV):
            peer = lax.rem(me + kk, N_DEV)
            pl.semaphore_signal(
                bar, inc=1,
                device_id=(peer,), device_id_type=pl.DeviceIdType.MESH,
            )
        pl.semaphore_wait(bar, N_DEV - 1)

        def w_copy(s, slot):
            j = lax.rem(me + _SHIFT_ORDER[s], N_DEV)
            return pltpu.make_async_copy(
                w_hbm.at[:, pl.ds(j * n_per, n_per)],
                w_buf.at[slot],
                copy_sems.at[slot],
            )

        def rdma_for(s, slot):
            peer = lax.rem(me + _SHIFT_ORDER[s], N_DEV)
            return pltpu.make_async_remote_copy(
                src_ref=send_buf.at[slot],
                dst_ref=recv_buf.at[s],
                send_sem=send_sems.at[s],
                recv_sem=recv_sems.at[s],
                device_id=(peer,),
                device_id_type=pl.DeviceIdType.MESH,
            )

        x_bf[:, :] = x_ref[:, :].astype(jnp.bfloat16)

        for s in range(_N_WSLOT - 1):
            w_copy(s, s % _N_WSLOT).start()

        for s in range(N_DEV):
            if s + _N_WSLOT - 1 < N_DEV:
                w_copy(s + _N_WSLOT - 1, (s + _N_WSLOT - 1) % _N_WSLOT).start()
            w_copy(s, s % _N_WSLOT).wait()

            y = _gelu(jnp.dot(x_bf[:, :],
                              w_buf[s % _N_WSLOT].astype(jnp.bfloat16),
                              preferred_element_type=jnp.float32))

            if s < N_DEV - 1:
                if s >= _N_SSLOT:
                    rdma_for(s - _N_SSLOT, s % _N_SSLOT).wait_send()
                send_buf[s % _N_SSLOT, :, :] = y.astype(jnp.bfloat16)
                rdma_for(s, s % _N_SSLOT).start()
            else:
                out_ref[pl.ds(me * m_per, m_per), :] = y

        for s in range(N_DEV - 1 - _N_SSLOT, N_DEV - 1):
            rdma_for(s, s % _N_SSLOT).wait_send()

        for s in range(N_DEV - 1):
            src_d = lax.rem(me + N_DEV - _SHIFT_ORDER[s], N_DEV)
            recv = pltpu.make_async_remote_copy(
                src_ref=send_buf.at[0],
                dst_ref=recv_buf.at[s],
                send_sem=send_sems.at[s],
                recv_sem=recv_sems.at[s],
                device_id=(0,),
                device_id_type=pl.DeviceIdType.MESH,
            )
            recv.wait_recv()
            out_ref[pl.ds(src_d * m_per, m_per), :] = (
                recv_buf[s, :, :].astype(jnp.float32))

        def _exit_barrier(second_bar):
            for kk in range(1, N_DEV):
                peer = lax.rem(me + kk, N_DEV)
                pl.semaphore_signal(
                    second_bar, inc=1,
                    device_id=(peer,), device_id_type=pl.DeviceIdType.MESH,
                )
            pl.semaphore_wait(second_bar, N_DEV - 1)

        pl.run_scoped(_exit_barrier, pltpu.SemaphoreType.REGULAR)

    return pl.pallas_call(
        body,
        out_shape=jax.ShapeDtypeStruct((N_DEV * m_per, n_per), jnp.float32),
        in_specs=[
            pl.BlockSpec(memory_space=pltpu.VMEM),
            pl.BlockSpec(memory_space=pl.ANY),
        ],
        out_specs=pl.BlockSpec(memory_space=pltpu.VMEM),
        scratch_shapes=[
            pltpu.VMEM((m_per, k_dim), jnp.bfloat16),
            pltpu.VMEM((_N_WSLOT, k_dim, n_per), jnp.float32),
            pltpu.VMEM((_N_SSLOT, m_per, n_per), jnp.bfloat16),
            pltpu.VMEM((N_DEV - 1, m_per, n_per), jnp.bfloat16),
            pltpu.SemaphoreType.DMA((_N_WSLOT,)),
            pltpu.SemaphoreType.DMA((N_DEV,)),
            pltpu.SemaphoreType.DMA((N_DEV,)),
        ],
        compiler_params=pltpu.CompilerParams(
            collective_id=0, vmem_limit_bytes=64 * 1024 * 1024),
    )(x, w_mat)


# device time: 72424 ns/iter; 1.2498x vs baseline; 1.2498x over previous
import jax
import jax.numpy as jnp
from jax import lax
from jax.experimental import pallas as pl
from jax.experimental.pallas import tpu as pltpu

N_DEV = 16
_GELU_C = 0.7978845608028654

_SHIFT_ORDER = [8, 7, 9, 6, 10, 5, 11, 4, 12, 3, 13, 2, 14, 1, 15, 0]

_N_WSLOT = 3
_N_SSLOT = N_DEV - 1


def _gelu(y):
    return 0.5 * y * (1.0 + jnp.tanh(_GELU_C * (y + 0.044715 * y * y * y)))


def kernel(x, w_mat):
    m_per, k_dim = x.shape
    _, n = w_mat.shape
    n_per = n // N_DEV

    def body(x_ref, w_hbm, out_ref, x_bf, w_buf, send_buf, recv_buf,
             copy_sems, send_sems, recv_sems):
        me = lax.axis_index("i")

        bar = pltpu.get_barrier_semaphore()
        for kk in range(1, N_DEV):
            peer = lax.rem(me + kk, N_DEV)
            pl.semaphore_signal(
                bar, inc=1,
                device_id=(peer,), device_id_type=pl.DeviceIdType.MESH,
            )
        pl.semaphore_wait(bar, N_DEV - 1)

        def w_copy(s, slot):
            j = lax.rem(me + _SHIFT_ORDER[s], N_DEV)
            return pltpu.make_async_copy(
                w_hbm.at[:, pl.ds(j * n_per, n_per)],
                w_buf.at[slot],
                copy_sems.at[slot],
            )

        def rdma_for(s, slot):
            peer = lax.rem(me + _SHIFT_ORDER[s], N_DEV)
            return pltpu.make_async_remote_copy(
                src_ref=send_buf.at[slot],
                dst_ref=recv_buf.at[s],
                send_sem=send_sems.at[s],
                recv_sem=recv_sems.at[s],
                device_id=(peer,),
                device_id_type=pl.DeviceIdType.MESH,
            )

        x_bf[:, :] = x_ref[:, :].astype(jnp.bfloat16)

        for s in range(_N_WSLOT - 1):
            w_copy(s, s % _N_WSLOT).start()

        for s in range(N_DEV):
            if s + _N_WSLOT - 1 < N_DEV:
                w_copy(s + _N_WSLOT - 1, (s + _N_WSLOT - 1) % _N_WSLOT).start()
            w_copy(s, s % _N_WSLOT).wait()

            y = _gelu(jnp.dot(x_bf[:, :],
                              w_buf[s % _N_WSLOT].astype(jnp.bfloat16),
                              preferred_element_type=jnp.float32))

            if s < N_DEV - 1:
                send_buf[s, :, :] = y.astype(jnp.bfloat16)
                rdma_for(s, s).start()
            else:
                out_ref[pl.ds(me * m_per, m_per), :] = y

        for s in range(N_DEV - 1):
            rdma_for(s, s).wait_send()

        for s in range(N_DEV - 1):
            src_d = lax.rem(me + N_DEV - _SHIFT_ORDER[s], N_DEV)
            recv = pltpu.make_async_remote_copy(
                src_ref=send_buf.at[0],
                dst_ref=recv_buf.at[s],
                send_sem=send_sems.at[s],
                recv_sem=recv_sems.at[s],
                device_id=(0,),
                device_id_type=pl.DeviceIdType.MESH,
            )
            recv.wait_recv()
            out_ref[pl.ds(src_d * m_per, m_per), :] = (
                recv_buf[s, :, :].astype(jnp.float32))

        def _exit_barrier(second_bar):
            for kk in range(1, N_DEV):
                peer = lax.rem(me + kk, N_DEV)
                pl.semaphore_signal(
                    second_bar, inc=1,
                    device_id=(peer,), device_id_type=pl.DeviceIdType.MESH,
                )
            pl.semaphore_wait(second_bar, N_DEV - 1)

        pl.run_scoped(_exit_barrier, pltpu.SemaphoreType.REGULAR)

    return pl.pallas_call(
        body,
        out_shape=jax.ShapeDtypeStruct((N_DEV * m_per, n_per), jnp.float32),
        in_specs=[
            pl.BlockSpec(memory_space=pltpu.VMEM),
            pl.BlockSpec(memory_space=pl.ANY),
        ],
        out_specs=pl.BlockSpec(memory_space=pltpu.VMEM),
        scratch_shapes=[
            pltpu.VMEM((m_per, k_dim), jnp.bfloat16),
            pltpu.VMEM((_N_WSLOT, k_dim, n_per), jnp.float32),
            pltpu.VMEM((_N_SSLOT, m_per, n_per), jnp.bfloat16),
            pltpu.VMEM((N_DEV - 1, m_per, n_per), jnp.bfloat16),
            pltpu.SemaphoreType.DMA((_N_WSLOT,)),
            pltpu.SemaphoreType.DMA((N_DEV,)),
            pltpu.SemaphoreType.DMA((N_DEV,)),
        ],
        compiler_params=pltpu.CompilerParams(
            collective_id=0, vmem_limit_bytes=64 * 1024 * 1024),
    )(x, w_mat)
